# Initial kernel scaffold; baseline (speedup 1.0000x reference)
import jax
import jax.numpy as jnp
from jax.experimental import pallas as pl


def kernel(x, pos, batch, norm, W1, b1, W2, b2, W3, b3):
    raise NotImplementedError("write your pallas kernel here")


# speedup vs baseline: 4.4647x; 4.4647x over previous
import jax
import jax.numpy as jnp
import numpy as np
from jax.experimental import pallas as pl
from jax.experimental.pallas import tpu as pltpu

_N = 10000
_NP = 10240
_ROWS = 8
_COLS = _NP // _ROWS
_NIN = 128
_NOUT = 256
_RADIUS = 0.2
_K = 32
_DIM = _NIN + 4
_NS = 2500


def _fps_kernel(px_ref, py_ref, pz_ref, idx_ref, cx_ref, cy_ref, cz_ref):
    px = px_ref[:, :]
    py = py_ref[:, :]
    pz = pz_ref[:, :]
    lin = (jax.lax.broadcasted_iota(jnp.int32, (_ROWS, _COLS), 0) * _COLS
           + jax.lax.broadcasted_iota(jnp.int32, (_ROWS, _COLS), 1))
    valid = lin < _N
    big = jnp.int32(2 ** 30)

    def extract(j, arr):
        return jnp.sum(jnp.where(lin == j, arr, 0.0))

    j0 = jnp.int32(0)
    lx0 = extract(j0, px)
    ly0 = extract(j0, py)
    lz0 = extract(j0, pz)
    idx_ref[0] = j0
    cx_ref[0] = lx0
    cy_ref[0] = ly0
    cz_ref[0] = lz0
    dists0 = jnp.where(valid, jnp.inf, -jnp.inf).astype(jnp.float32)

    def body(i, carry):
        dists, lx, ly, lz = carry
        dx = px - lx
        dy = py - ly
        dz = pz - lz
        d = dx * dx + dy * dy + dz * dz
        dists = jnp.minimum(dists, d)
        m = jnp.max(dists)
        j = jnp.min(jnp.where(dists == m, lin, big))
        nx = extract(j, px)
        ny = extract(j, py)
        nz = extract(j, pz)
        idx_ref[i] = j
        cx_ref[i] = nx
        cy_ref[i] = ny
        cz_ref[i] = nz
        return (dists, nx, ny, nz)

    jax.lax.fori_loop(1, _NS, body, (dists0, lx0, ly0, lz0))


def _run_fps(pos):
    posp = jnp.pad(pos, ((0, _NP - _N), (0, 0)))
    px = posp[:, 0].reshape(_ROWS, _COLS)
    py = posp[:, 1].reshape(_ROWS, _COLS)
    pz = posp[:, 2].reshape(_ROWS, _COLS)
    idx, cx, cy, cz = pl.pallas_call(
        _fps_kernel,
        out_shape=(jax.ShapeDtypeStruct((_NS,), jnp.int32),
                   jax.ShapeDtypeStruct((_NS,), jnp.float32),
                   jax.ShapeDtypeStruct((_NS,), jnp.float32),
                   jax.ShapeDtypeStruct((_NS,), jnp.float32)),
        out_specs=(pl.BlockSpec(memory_space=pltpu.SMEM),) * 4,
    )(px, py, pz)
    return idx, jnp.stack([cx, cy, cz], axis=-1)


def _safe_norm(v, axis=-1):
    return jnp.sqrt(jnp.sum(v * v, axis=axis) + 1e-12)


def _get_angle(v1, v2):
    cross = jnp.cross(v1, v2)
    return jnp.arctan2(_safe_norm(cross), jnp.sum(v1 * v2, axis=-1))


def kernel(x, pos, batch, norm, W1, b1, W2, b2, W3, b3):
    idx, centers = _run_fps(pos)

    d2 = (jnp.sum(centers * centers, axis=1)[:, None]
          + jnp.sum(pos * pos, axis=1)[None, :]
          - 2.0 * (centers @ pos.T))
    d2 = jnp.maximum(d2, 0.0)
    mask = d2 <= _RADIUS * _RADIUS
    d2m = jnp.where(mask, d2, jnp.inf)
    negv, nbr = jax.lax.top_k(-d2m, _K)
    valid = jnp.isfinite(negv)

    pos_i = centers[:, None, :]
    pos_j = pos[nbr]
    n_i = jnp.broadcast_to(norm[idx][:, None, :], pos_j.shape)
    n_j = norm[nbr]
    pseudo = pos_j - pos_i
    ppf = jnp.stack([
        _safe_norm(pseudo),
        _get_angle(n_i, pseudo),
        _get_angle(n_j, pseudo),
        _get_angle(n_i, n_j),
    ], axis=-1)
    x_j = x[nbr]
    h = jnp.concatenate([x_j, ppf], axis=-1)
    h = jax.nn.relu(h @ W1 + b1)
    h = jax.nn.relu(h @ W2 + b2)
    msg = jnp.where(valid[:, :, None], h, -jnp.inf)
    agg = jnp.max(msg, axis=1)
    out = jax.nn.relu(agg @ W3 + b3)
    return (out, centers, batch[idx], idx)


# speedup vs baseline: 6.7079x; 1.5024x over previous
import jax
import jax.numpy as jnp
import numpy as np
from jax.experimental import pallas as pl
from jax.experimental.pallas import tpu as pltpu

_N = 10000
_NP = 10240
_ROWS = 8
_COLS = _NP // _ROWS
_NIN = 128
_NOUT = 256
_RADIUS = 0.2
_K = 32
_DIM = _NIN + 4
_NS = 2500


def _fps_kernel(px_ref, py_ref, pz_ref, idx_ref, cx_ref, cy_ref, cz_ref):
    px = px_ref[:, :]
    py = py_ref[:, :]
    pz = pz_ref[:, :]
    lin = (jax.lax.broadcasted_iota(jnp.int32, (_ROWS, _COLS), 0) * _COLS
           + jax.lax.broadcasted_iota(jnp.int32, (_ROWS, _COLS), 1))
    valid = lin < _N
    big = jnp.int32(2 ** 30)

    def extract(j, arr):
        return jnp.sum(jnp.where(lin == j, arr, 0.0))

    j0 = jnp.int32(0)
    lx0 = extract(j0, px)
    ly0 = extract(j0, py)
    lz0 = extract(j0, pz)
    idx_ref[0] = j0
    cx_ref[0] = lx0
    cy_ref[0] = ly0
    cz_ref[0] = lz0
    dists0 = jnp.where(valid, jnp.inf, -jnp.inf).astype(jnp.float32)

    def body(i, carry):
        dists, lx, ly, lz = carry
        dx = px - lx
        dy = py - ly
        dz = pz - lz
        d = dx * dx + dy * dy + dz * dz
        dists = jnp.minimum(dists, d)
        m = jnp.max(dists)
        j = jnp.min(jnp.where(dists == m, lin, big))
        nx = extract(j, px)
        ny = extract(j, py)
        nz = extract(j, pz)
        idx_ref[i] = j
        cx_ref[i] = nx
        cy_ref[i] = ny
        cz_ref[i] = nz
        return (dists, nx, ny, nz)

    jax.lax.fori_loop(1, _NS, body, (dists0, lx0, ly0, lz0))


def _run_fps(pos):
    posp = jnp.pad(pos, ((0, _NP - _N), (0, 0)))
    px = posp[:, 0].reshape(_ROWS, _COLS)
    py = posp[:, 1].reshape(_ROWS, _COLS)
    pz = posp[:, 2].reshape(_ROWS, _COLS)
    idx, cx, cy, cz = pl.pallas_call(
        _fps_kernel,
        out_shape=(jax.ShapeDtypeStruct((_NS,), jnp.int32),
                   jax.ShapeDtypeStruct((_NS,), jnp.float32),
                   jax.ShapeDtypeStruct((_NS,), jnp.float32),
                   jax.ShapeDtypeStruct((_NS,), jnp.float32)),
        out_specs=(pl.BlockSpec(memory_space=pltpu.SMEM),) * 4,
    )(px, py, pz)
    return idx, jnp.stack([cx, cy, cz], axis=-1)


_NSP = 2512
_CB = 16
_NBLK = _NSP // _CB
_R2 = np.float32(0.2) * np.float32(0.2)


def _nbr_kernel(c_ref, px_ref, py_ref, pz_ref, nbr_ref, val_ref):
    c = c_ref[...]
    cx = c[:, 0:1]
    cy = c[:, 1:2]
    cz = c[:, 2:3]
    px = px_ref[...]
    py = py_ref[...]
    pz = pz_ref[...]
    dx = cx - px
    dy = cy - py
    dz = cz - pz
    d2 = dx * dx + dy * dy + dz * dz
    d2m = jnp.where(d2 <= _R2, d2, jnp.inf)
    lin = jax.lax.broadcasted_iota(jnp.int32, (_CB, _NP), 1)
    lane32 = jax.lax.broadcasted_iota(jnp.int32, (_CB, _K), 1)
    big = jnp.int32(2 ** 30)

    def body(t, carry):
        d2m, nbr_acc, val_acc = carry
        m = jnp.min(d2m, axis=1, keepdims=True)
        sel = jnp.min(jnp.where(d2m == m, lin, big), axis=1, keepdims=True)
        v = (m <= _R2).astype(jnp.int32)
        nbr_acc = jnp.where(lane32 == t, sel, nbr_acc)
        val_acc = jnp.where(lane32 == t, v, val_acc)
        d2m = jnp.where(lin == sel, jnp.inf, d2m)
        return (d2m, nbr_acc, val_acc)

    zeros = jnp.zeros((_CB, _K), jnp.int32)
    _, nbr_acc, val_acc = jax.lax.fori_loop(0, _K, body, (d2m, zeros, zeros))
    nbr_ref[...] = nbr_acc
    val_ref[...] = val_acc


def _run_nbr(centers, pos):
    cp = jnp.full((_NSP, 3), 1e9, jnp.float32).at[:_NS].set(centers)
    posp = jnp.pad(pos, ((0, _NP - _N), (0, 0)), constant_values=100.0)
    px = posp[:, 0].reshape(1, _NP)
    py = posp[:, 1].reshape(1, _NP)
    pz = posp[:, 2].reshape(1, _NP)
    nbr, val = pl.pallas_call(
        _nbr_kernel,
        grid=(_NBLK,),
        in_specs=[
            pl.BlockSpec((_CB, 3), lambda i: (i, 0)),
            pl.BlockSpec((1, _NP), lambda i: (0, 0)),
            pl.BlockSpec((1, _NP), lambda i: (0, 0)),
            pl.BlockSpec((1, _NP), lambda i: (0, 0)),
        ],
        out_specs=(pl.BlockSpec((_CB, _K), lambda i: (i, 0)),
                   pl.BlockSpec((_CB, _K), lambda i: (i, 0))),
        out_shape=(jax.ShapeDtypeStruct((_NSP, _K), jnp.int32),
                   jax.ShapeDtypeStruct((_NSP, _K), jnp.int32)),
    )(cp, px, py, pz)
    return nbr[:_NS], val[:_NS]


def _safe_norm(v, axis=-1):
    return jnp.sqrt(jnp.sum(v * v, axis=axis) + 1e-12)


def _get_angle(v1, v2):
    cross = jnp.cross(v1, v2)
    return jnp.arctan2(_safe_norm(cross), jnp.sum(v1 * v2, axis=-1))


def kernel(x, pos, batch, norm, W1, b1, W2, b2, W3, b3):
    idx, centers = _run_fps(pos)

    nbr, vali = _run_nbr(centers, pos)
    valid = vali.astype(bool)

    pos_i = centers[:, None, :]
    pos_j = pos[nbr]
    n_i = jnp.broadcast_to(norm[idx][:, None, :], pos_j.shape)
    n_j = norm[nbr]
    pseudo = pos_j - pos_i
    ppf = jnp.stack([
        _safe_norm(pseudo),
        _get_angle(n_i, pseudo),
        _get_angle(n_j, pseudo),
        _get_angle(n_i, n_j),
    ], axis=-1)
    x_j = x[nbr]
    h = jnp.concatenate([x_j, ppf], axis=-1)
    h = jax.nn.relu(h @ W1 + b1)
    h = jax.nn.relu(h @ W2 + b2)
    msg = jnp.where(valid[:, :, None], h, -jnp.inf)
    agg = jnp.max(msg, axis=1)
    out = jax.nn.relu(agg @ W3 + b3)
    return (out, centers, batch[idx], idx)


# speedup vs baseline: 6.7762x; 1.0102x over previous
import jax
import jax.numpy as jnp
import numpy as np
from jax.experimental import pallas as pl
from jax.experimental.pallas import tpu as pltpu

_N = 10000
_NP = 10240
_ROWS = 8
_COLS = _NP // _ROWS
_NIN = 128
_NOUT = 256
_RADIUS = 0.2
_K = 32
_DIM = _NIN + 4
_NS = 2500


def _fps_kernel(px_ref, py_ref, pz_ref, idx_ref, cx_ref, cy_ref, cz_ref):
    px = px_ref[:, :]
    py = py_ref[:, :]
    pz = pz_ref[:, :]
    lin = (jax.lax.broadcasted_iota(jnp.int32, (_ROWS, _COLS), 0) * _COLS
           + jax.lax.broadcasted_iota(jnp.int32, (_ROWS, _COLS), 1))
    valid = lin < _N
    big = jnp.int32(2 ** 30)

    def extract(j, arr):
        return jnp.sum(jnp.where(lin == j, arr, 0.0))

    j0 = jnp.int32(0)
    lx0 = extract(j0, px)
    ly0 = extract(j0, py)
    lz0 = extract(j0, pz)
    idx_ref[0] = j0
    cx_ref[0] = lx0
    cy_ref[0] = ly0
    cz_ref[0] = lz0
    dists0 = jnp.where(valid, jnp.inf, -jnp.inf).astype(jnp.float32)

    def body(i, carry):
        dists, lx, ly, lz = carry
        dx = px - lx
        dy = py - ly
        dz = pz - lz
        d = dx * dx + dy * dy + dz * dz
        dists = jnp.minimum(dists, d)
        m = jnp.max(dists)
        j = jnp.min(jnp.where(dists == m, lin, big))
        nx = extract(j, px)
        ny = extract(j, py)
        nz = extract(j, pz)
        idx_ref[i] = j
        cx_ref[i] = nx
        cy_ref[i] = ny
        cz_ref[i] = nz
        return (dists, nx, ny, nz)

    jax.lax.fori_loop(1, _NS, body, (dists0, lx0, ly0, lz0))


def _run_fps(pos):
    posp = jnp.pad(pos, ((0, _NP - _N), (0, 0)))
    px = posp[:, 0].reshape(_ROWS, _COLS)
    py = posp[:, 1].reshape(_ROWS, _COLS)
    pz = posp[:, 2].reshape(_ROWS, _COLS)
    idx, cx, cy, cz = pl.pallas_call(
        _fps_kernel,
        out_shape=(jax.ShapeDtypeStruct((_NS,), jnp.int32),
                   jax.ShapeDtypeStruct((_NS,), jnp.float32),
                   jax.ShapeDtypeStruct((_NS,), jnp.float32),
                   jax.ShapeDtypeStruct((_NS,), jnp.float32)),
        out_specs=(pl.BlockSpec(memory_space=pltpu.SMEM),) * 4,
    )(px, py, pz)
    return idx, jnp.stack([cx, cy, cz], axis=-1)


_NSP = 2512
_CB = 16
_NBLK = _NSP // _CB
_R2 = np.float32(0.2 * 0.2)


def _nbr_kernel(c_ref, pt_ref, nbr_ref, val_ref):
    c = c_ref[...]
    pt = pt_ref[...]
    cc = jnp.sum(c * c, axis=1, keepdims=True)
    pp = jnp.sum(pt * pt, axis=0, keepdims=True)
    mm = jax.lax.dot_general(c, pt, (((1,), (0,)), ((), ())),
                             preferred_element_type=jnp.float32)
    d2 = cc + pp - 2.0 * mm
    d2 = jnp.maximum(d2, 0.0)
    d2m = jnp.where(d2 <= _R2, d2, jnp.inf)
    lin = jax.lax.broadcasted_iota(jnp.int32, (_CB, _NP), 1)
    lane32 = jax.lax.broadcasted_iota(jnp.int32, (_CB, _K), 1)
    big = jnp.int32(2 ** 30)

    def body(t, carry):
        d2m, nbr_acc, val_acc = carry
        m = jnp.min(d2m, axis=1, keepdims=True)
        sel = jnp.min(jnp.where(d2m == m, lin, big), axis=1, keepdims=True)
        v = (m <= _R2).astype(jnp.int32)
        nbr_acc = jnp.where(lane32 == t, sel, nbr_acc)
        val_acc = jnp.where(lane32 == t, v, val_acc)
        d2m = jnp.where(lin == sel, jnp.inf, d2m)
        return (d2m, nbr_acc, val_acc)

    zeros = jnp.zeros((_CB, _K), jnp.int32)
    _, nbr_acc, val_acc = jax.lax.fori_loop(0, _K, body, (d2m, zeros, zeros))
    nbr_ref[...] = nbr_acc
    val_ref[...] = val_acc


def _run_nbr(centers, pos):
    cp = jnp.full((_NSP, 3), 1e9, jnp.float32).at[:_NS].set(centers)
    posp = jnp.pad(pos, ((0, _NP - _N), (0, 0)), constant_values=100.0)
    pt = posp.T
    nbr, val = pl.pallas_call(
        _nbr_kernel,
        grid=(_NBLK,),
        in_specs=[
            pl.BlockSpec((_CB, 3), lambda i: (i, 0)),
            pl.BlockSpec((3, _NP), lambda i: (0, 0)),
        ],
        out_specs=(pl.BlockSpec((_CB, _K), lambda i: (i, 0)),
                   pl.BlockSpec((_CB, _K), lambda i: (i, 0))),
        out_shape=(jax.ShapeDtypeStruct((_NSP, _K), jnp.int32),
                   jax.ShapeDtypeStruct((_NSP, _K), jnp.int32)),
    )(cp, pt)
    return nbr[:_NS], val[:_NS]


def _safe_norm(v, axis=-1):
    return jnp.sqrt(jnp.sum(v * v, axis=axis) + 1e-12)


def _get_angle(v1, v2):
    cross = jnp.cross(v1, v2)
    return jnp.arctan2(_safe_norm(cross), jnp.sum(v1 * v2, axis=-1))


def kernel(x, pos, batch, norm, W1, b1, W2, b2, W3, b3):
    idx, centers = _run_fps(pos)

    nbr, vali = _run_nbr(centers, pos)
    valid = vali.astype(bool)

    pos_i = centers[:, None, :]
    pos_j = pos[nbr]
    n_i = jnp.broadcast_to(norm[idx][:, None, :], pos_j.shape)
    n_j = norm[nbr]
    pseudo = pos_j - pos_i
    ppf = jnp.stack([
        _safe_norm(pseudo),
        _get_angle(n_i, pseudo),
        _get_angle(n_j, pseudo),
        _get_angle(n_i, n_j),
    ], axis=-1)
    x_j = x[nbr]
    h = jnp.concatenate([x_j, ppf], axis=-1)
    h = jax.nn.relu(h @ W1 + b1)
    h = jax.nn.relu(h @ W2 + b2)
    msg = jnp.where(valid[:, :, None], h, -jnp.inf)
    agg = jnp.max(msg, axis=1)
    out = jax.nn.relu(agg @ W3 + b3)
    return (out, centers, batch[idx], idx)


# speedup vs baseline: 7.3063x; 1.0782x over previous
import jax
import jax.numpy as jnp
import numpy as np
from jax.experimental import pallas as pl
from jax.experimental.pallas import tpu as pltpu

_N = 10000
_NP = 10240
_ROWS = 8
_COLS = _NP // _ROWS
_NIN = 128
_NOUT = 256
_RADIUS = 0.2
_K = 32
_DIM = _NIN + 4
_NS = 2500


def _fps_kernel(px_ref, py_ref, pz_ref, sx_ref, sy_ref, sz_ref,
                idx_ref, cx_ref, cy_ref, cz_ref):
    px = px_ref[:, :]
    py = py_ref[:, :]
    pz = pz_ref[:, :]
    lin = (jax.lax.broadcasted_iota(jnp.int32, (_ROWS, _COLS), 0) * _COLS
           + jax.lax.broadcasted_iota(jnp.int32, (_ROWS, _COLS), 1))
    valid = lin < _N
    big = jnp.int32(2 ** 30)

    idx_ref[0] = jnp.int32(0)
    lx0 = sx_ref[0]
    ly0 = sy_ref[0]
    lz0 = sz_ref[0]
    cx_ref[0] = lx0
    cy_ref[0] = ly0
    cz_ref[0] = lz0
    dists0 = jnp.where(valid, jnp.inf, -jnp.inf).astype(jnp.float32)

    def body(i, carry):
        dists, lx, ly, lz = carry
        dx = px - lx
        dy = py - ly
        dz = pz - lz
        d = dx * dx + dy * dy + dz * dz
        dists = jnp.minimum(dists, d)
        m = jnp.max(dists)
        j = jnp.min(jnp.where(dists == m, lin, big))
        nx = sx_ref[j]
        ny = sy_ref[j]
        nz = sz_ref[j]
        idx_ref[i] = j
        cx_ref[i] = nx
        cy_ref[i] = ny
        cz_ref[i] = nz
        return (dists, nx, ny, nz)

    jax.lax.fori_loop(1, _NS, body, (dists0, lx0, ly0, lz0))


def _run_fps(pos):
    posp = jnp.pad(pos, ((0, _NP - _N), (0, 0)))
    px = posp[:, 0].reshape(_ROWS, _COLS)
    py = posp[:, 1].reshape(_ROWS, _COLS)
    pz = posp[:, 2].reshape(_ROWS, _COLS)
    sx = posp[:, 0]
    sy = posp[:, 1]
    sz = posp[:, 2]
    idx, cx, cy, cz = pl.pallas_call(
        _fps_kernel,
        in_specs=[
            pl.BlockSpec((_ROWS, _COLS), lambda: (0, 0)),
            pl.BlockSpec((_ROWS, _COLS), lambda: (0, 0)),
            pl.BlockSpec((_ROWS, _COLS), lambda: (0, 0)),
            pl.BlockSpec(memory_space=pltpu.SMEM),
            pl.BlockSpec(memory_space=pltpu.SMEM),
            pl.BlockSpec(memory_space=pltpu.SMEM),
        ],
        out_shape=(jax.ShapeDtypeStruct((_NS,), jnp.int32),
                   jax.ShapeDtypeStruct((_NS,), jnp.float32),
                   jax.ShapeDtypeStruct((_NS,), jnp.float32),
                   jax.ShapeDtypeStruct((_NS,), jnp.float32)),
        out_specs=(pl.BlockSpec(memory_space=pltpu.SMEM),) * 4,
    )(px, py, pz, sx, sy, sz)
    return idx, jnp.stack([cx, cy, cz], axis=-1)


_NSP = 2512
_CB = 16
_NBLK = _NSP // _CB
_R2 = np.float32(0.2 * 0.2)


def _nbr_kernel(c_ref, pt_ref, nbr_ref, val_ref):
    c = c_ref[...]
    pt = pt_ref[...]
    cc = jnp.sum(c * c, axis=1, keepdims=True)
    pp = jnp.sum(pt * pt, axis=0, keepdims=True)
    mm = jax.lax.dot_general(c, pt, (((1,), (0,)), ((), ())),
                             preferred_element_type=jnp.float32)
    d2 = cc + pp - 2.0 * mm
    d2 = jnp.maximum(d2, 0.0)
    d2m = jnp.where(d2 <= _R2, d2, jnp.inf)
    lin = jax.lax.broadcasted_iota(jnp.int32, (_CB, _NP), 1)
    lane32 = jax.lax.broadcasted_iota(jnp.int32, (_CB, _K), 1)
    big = jnp.int32(2 ** 30)

    def body(t, carry):
        d2m, nbr_acc, val_acc = carry
        m = jnp.min(d2m, axis=1, keepdims=True)
        sel = jnp.min(jnp.where(d2m == m, lin, big), axis=1, keepdims=True)
        v = (m <= _R2).astype(jnp.int32)
        nbr_acc = jnp.where(lane32 == t, sel, nbr_acc)
        val_acc = jnp.where(lane32 == t, v, val_acc)
        d2m = jnp.where(lin == sel, jnp.inf, d2m)
        return (d2m, nbr_acc, val_acc)

    zeros = jnp.zeros((_CB, _K), jnp.int32)
    _, nbr_acc, val_acc = jax.lax.fori_loop(0, _K, body, (d2m, zeros, zeros))
    nbr_ref[...] = nbr_acc
    val_ref[...] = val_acc


def _run_nbr(centers, pos):
    cp = jnp.full((_NSP, 3), 1e9, jnp.float32).at[:_NS].set(centers)
    posp = jnp.pad(pos, ((0, _NP - _N), (0, 0)), constant_values=100.0)
    pt = posp.T
    nbr, val = pl.pallas_call(
        _nbr_kernel,
        grid=(_NBLK,),
        in_specs=[
            pl.BlockSpec((_CB, 3), lambda i: (i, 0)),
            pl.BlockSpec((3, _NP), lambda i: (0, 0)),
        ],
        out_specs=(pl.BlockSpec((_CB, _K), lambda i: (i, 0)),
                   pl.BlockSpec((_CB, _K), lambda i: (i, 0))),
        out_shape=(jax.ShapeDtypeStruct((_NSP, _K), jnp.int32),
                   jax.ShapeDtypeStruct((_NSP, _K), jnp.int32)),
    )(cp, pt)
    return nbr[:_NS], val[:_NS]


def _safe_norm(v, axis=-1):
    return jnp.sqrt(jnp.sum(v * v, axis=axis) + 1e-12)


def _get_angle(v1, v2):
    cross = jnp.cross(v1, v2)
    return jnp.arctan2(_safe_norm(cross), jnp.sum(v1 * v2, axis=-1))


def kernel(x, pos, batch, norm, W1, b1, W2, b2, W3, b3):
    idx, centers = _run_fps(pos)

    nbr, vali = _run_nbr(centers, pos)
    valid = vali.astype(bool)

    pos_i = centers[:, None, :]
    pos_j = pos[nbr]
    n_i = jnp.broadcast_to(norm[idx][:, None, :], pos_j.shape)
    n_j = norm[nbr]
    pseudo = pos_j - pos_i
    ppf = jnp.stack([
        _safe_norm(pseudo),
        _get_angle(n_i, pseudo),
        _get_angle(n_j, pseudo),
        _get_angle(n_i, n_j),
    ], axis=-1)
    x_j = x[nbr]
    h = jnp.concatenate([x_j, ppf], axis=-1)
    h = jax.nn.relu(h @ W1 + b1)
    h = jax.nn.relu(h @ W2 + b2)
    msg = jnp.where(valid[:, :, None], h, -jnp.inf)
    agg = jnp.max(msg, axis=1)
    out = jax.nn.relu(agg @ W3 + b3)
    return (out, centers, batch[idx], idx)
